# Initial kernel scaffold; baseline (speedup 1.0000x reference)
import jax
import jax.numpy as jnp
from jax.experimental import pallas as pl


def kernel(features, caption, emb, W_ih, W_hh, b_ih, b_hh, W_lin, b_lin):
    raise NotImplementedError("write your pallas kernel here")


# speedup vs baseline: 2.4218x; 2.4218x over previous
import functools

import jax
import jax.numpy as jnp
from jax import lax
from jax.experimental import pallas as pl
from jax.experimental.pallas import tpu as pltpu
from jax.experimental.pallas import tpu_sc as plsc

VOCAB = 100000
EMB = 64
HID = 256
CAP_LEN = 199
SEQ = CAP_LEN + 1

_NW = 32
_B_PAD = 256
_B_PER_W = _B_PAD // _NW
_WIDE = 2 * EMB


@functools.lru_cache(maxsize=1)
def _sc_gather_build():
    mesh = plsc.VectorSubcoreMesh(core_axis_name="c", subcore_axis_name="s")

    @functools.partial(
        pl.kernel,
        mesh=mesh,
        out_type=jax.ShapeDtypeStruct((_B_PAD, _WIDE), jnp.float32),
        scratch_types=[
            pltpu.VMEM((_B_PER_W,), jnp.int32),
            pltpu.VMEM((_B_PER_W, _WIDE), jnp.float32),
            pltpu.SemaphoreType.DMA,
        ],
    )
    def gather_k(emb_hbm, idx_hbm, out_hbm, idx_v, rows_v, sem):
        wid = lax.axis_index("s") * 2 + lax.axis_index("c")
        base = wid * _B_PER_W
        pltpu.sync_copy(idx_hbm.at[pl.ds(base, _B_PER_W)], idx_v)
        pltpu.async_copy(emb_hbm.at[idx_v], rows_v, sem).wait()
        pltpu.sync_copy(rows_v, out_hbm.at[pl.ds(base, _B_PER_W)])

    return gather_k


def _gru_body(x_ref, wih_ref, whh_ref, bih_ref, bhh_ref, out_ref, gi_ref):
    gi_ref[:] = (
        jnp.dot(x_ref[:], wih_ref[:], preferred_element_type=jnp.float32)
        + bih_ref[:]
    )
    whh = whh_ref[:]
    bhh = bhh_ref[:]

    def step(t, h):
        gi = gi_ref[pl.ds(t, 1), :]
        gh = jnp.dot(h, whh, preferred_element_type=jnp.float32) + bhh
        i_r, i_z, i_n = gi[:, :HID], gi[:, HID:2 * HID], gi[:, 2 * HID:]
        h_r, h_z, h_n = gh[:, :HID], gh[:, HID:2 * HID], gh[:, 2 * HID:]
        r = jax.nn.sigmoid(i_r + h_r)
        z = jax.nn.sigmoid(i_z + h_z)
        n = jnp.tanh(i_n + r * h_n)
        h_new = (1.0 - z) * n + z * h
        out_ref[pl.ds(t, 1), :] = h_new
        return h_new

    lax.fori_loop(0, SEQ, step, jnp.zeros((1, HID), jnp.float32))


def _gru_call(x, wih_t, whh_t, bih, bhh):
    return pl.pallas_call(
        _gru_body,
        out_shape=jax.ShapeDtypeStruct((SEQ, HID), jnp.float32),
        scratch_shapes=[pltpu.VMEM((SEQ, 3 * HID), jnp.float32)],
    )(x, wih_t, whh_t, bih, bhh)


_BV = 2048


def _proj_body(g_ref, w_ref, b_ref, out_ref):
    out_ref[:] = (
        lax.dot_general(
            g_ref[:], w_ref[:],
            (((1,), (1,)), ((), ())),
            preferred_element_type=jnp.float32,
        )
        + b_ref[:]
    )


def _proj_call(gru_out, w_lin, b_lin2d):
    grid = (pl.cdiv(VOCAB, _BV),)
    return pl.pallas_call(
        _proj_body,
        grid=grid,
        in_specs=[
            pl.BlockSpec((SEQ, HID), lambda i: (0, 0)),
            pl.BlockSpec((_BV, HID), lambda i: (i, 0)),
            pl.BlockSpec((1, _BV), lambda i: (0, i)),
        ],
        out_specs=pl.BlockSpec((SEQ, _BV), lambda i: (0, i)),
        out_shape=jax.ShapeDtypeStruct((SEQ, VOCAB), jnp.float32),
    )(gru_out, w_lin, b_lin2d)


def kernel(features, caption, emb, W_ih, W_hh, b_ih, b_hh, W_lin, b_lin):
    cap32 = caption.astype(jnp.int32)
    cap_pad = jnp.zeros((_B_PAD,), jnp.int32).at[:CAP_LEN].set(cap32)
    emb_wide = emb.reshape(VOCAB // 2, _WIDE)
    wide = _sc_gather_build()(emb_wide, cap_pad >> 1)
    odd = (cap_pad[:CAP_LEN] & 1)[:, None].astype(jnp.bool_)
    embeds = jnp.where(odd, wide[:CAP_LEN, EMB:], wide[:CAP_LEN, :EMB])
    x = jnp.concatenate([features, embeds], axis=0)
    gru_out = _gru_call(x, W_ih.T, W_hh.T, b_ih[None, :], b_hh[None, :])
    return _proj_call(gru_out, W_lin, b_lin[None, :])


# speedup vs baseline: 2.4514x; 1.0122x over previous
import functools

import jax
import jax.numpy as jnp
from jax import lax
from jax.experimental import pallas as pl
from jax.experimental.pallas import tpu as pltpu
from jax.experimental.pallas import tpu_sc as plsc

VOCAB = 100000
EMB = 64
HID = 256
CAP_LEN = 199
SEQ = CAP_LEN + 1

_NW = 32
_B_PAD = 256
_B_PER_W = _B_PAD // _NW


@functools.lru_cache(maxsize=1)
def _sc_gather_build():
    mesh = plsc.VectorSubcoreMesh(core_axis_name="c", subcore_axis_name="s")

    @functools.partial(
        pl.kernel,
        mesh=mesh,
        out_type=jax.ShapeDtypeStruct((_B_PAD, EMB), jnp.float32),
        scratch_types=[
            pltpu.VMEM((_B_PER_W,), jnp.int32),
            pltpu.VMEM((_B_PER_W, EMB), jnp.float32),
            pltpu.SemaphoreType.DMA,
        ],
        compiler_params=pltpu.CompilerParams(use_tc_tiling_on_sc=False),
    )
    def gather_k(emb_hbm, idx_hbm, out_hbm, idx_v, rows_v, sem):
        wid = lax.axis_index("s") * 2 + lax.axis_index("c")
        base = wid * _B_PER_W
        pltpu.sync_copy(idx_hbm.at[pl.ds(base, _B_PER_W)], idx_v)
        pltpu.async_copy(emb_hbm.at[idx_v], rows_v, sem).wait()
        pltpu.sync_copy(rows_v, out_hbm.at[pl.ds(base, _B_PER_W)])

    return gather_k


def _gru_body(x_ref, wih_ref, whh_ref, bih_ref, bhh_ref, out_ref, gi_ref):
    gi_ref[:] = (
        jnp.dot(x_ref[:], wih_ref[:], preferred_element_type=jnp.float32)
        + bih_ref[:]
    )
    whh = whh_ref[:]
    bhh = bhh_ref[:]

    def step(t, h):
        gi = gi_ref[pl.ds(t, 1), :]
        gh = jnp.dot(h, whh, preferred_element_type=jnp.float32) + bhh
        i_r, i_z, i_n = gi[:, :HID], gi[:, HID:2 * HID], gi[:, 2 * HID:]
        h_r, h_z, h_n = gh[:, :HID], gh[:, HID:2 * HID], gh[:, 2 * HID:]
        r = jax.nn.sigmoid(i_r + h_r)
        z = jax.nn.sigmoid(i_z + h_z)
        n = jnp.tanh(i_n + r * h_n)
        h_new = (1.0 - z) * n + z * h
        out_ref[pl.ds(t, 1), :] = h_new
        return h_new

    lax.fori_loop(0, SEQ, step, jnp.zeros((1, HID), jnp.float32))


def _gru_call(x, wih_t, whh_t, bih, bhh):
    return pl.pallas_call(
        _gru_body,
        out_shape=jax.ShapeDtypeStruct((SEQ, HID), jnp.float32),
        scratch_shapes=[pltpu.VMEM((SEQ, 3 * HID), jnp.float32)],
    )(x, wih_t, whh_t, bih, bhh)


_BV = 2048


def _proj_body(g_ref, w_ref, b_ref, out_ref):
    out_ref[:] = (
        lax.dot_general(
            g_ref[:], w_ref[:],
            (((1,), (1,)), ((), ())),
            preferred_element_type=jnp.float32,
        )
        + b_ref[:]
    )


def _proj_call(gru_out, w_lin, b_lin2d):
    grid = (pl.cdiv(VOCAB, _BV),)
    return pl.pallas_call(
        _proj_body,
        grid=grid,
        in_specs=[
            pl.BlockSpec((SEQ, HID), lambda i: (0, 0)),
            pl.BlockSpec((_BV, HID), lambda i: (i, 0)),
            pl.BlockSpec((1, _BV), lambda i: (0, i)),
        ],
        out_specs=pl.BlockSpec((SEQ, _BV), lambda i: (0, i)),
        out_shape=jax.ShapeDtypeStruct((SEQ, VOCAB), jnp.float32),
    )(gru_out, w_lin, b_lin2d)


def kernel(features, caption, emb, W_ih, W_hh, b_ih, b_hh, W_lin, b_lin):
    cap32 = caption.astype(jnp.int32)
    cap_pad = jnp.zeros((_B_PAD,), jnp.int32).at[:CAP_LEN].set(cap32)
    embeds = _sc_gather_build()(emb, cap_pad)
    x = jnp.concatenate([features, embeds[:CAP_LEN]], axis=0)
    gru_out = _gru_call(x, W_ih.T, W_hh.T, b_ih[None, :], b_hh[None, :])
    return _proj_call(gru_out, W_lin, b_lin[None, :])
